# baseline (speedup 1.0000x reference)
import functools

import jax
import jax.numpy as jnp
from jax import lax
from jax.experimental import pallas as pl
from jax.experimental.pallas import tpu as pltpu
from jax.experimental.pallas import tpu_sc as plsc

N_NODES = 50000
B = 50000
K = 10
D = 128
E = 128
NW = 32
GN = 24
NSTREAM = 2
S_STEPS = 66
B_PAD = NW * S_STEPS * GN

RPS = GN * K // NSTREAM
FN = 2 * GN


def _prep_idx(pos_neigh, neg_neigh):
    def one(neigh):
        flat = neigh.astype(jnp.int32).reshape(-1)
        flat = jnp.pad(flat, (0, B_PAD * K - B * K))
        return flat.reshape(NW, S_STEPS, 1, NSTREAM, RPS)
    return jnp.concatenate([one(pos_neigh), one(neg_neigh)], axis=2)


def _sc_gather_sum(features, idx):
    info = plsc.get_sparse_core_info()
    nc = info.num_cores

    mesh = plsc.VectorSubcoreMesh(core_axis_name="c", subcore_axis_name="s")

    @functools.partial(
        pl.kernel,
        out_type=(jax.ShapeDtypeStruct((B_PAD, D), jnp.float32),
                  jax.ShapeDtypeStruct((B_PAD, D), jnp.float32)),
        mesh=mesh,
        scratch_types=[
            pltpu.VMEM((S_STEPS, 2, NSTREAM, RPS), jnp.int32),
            pltpu.VMEM((2, GN * K, D), jnp.float32),
            pltpu.VMEM((2, 2, FN, D), jnp.float32),
            pltpu.SemaphoreType.DMA,
            pltpu.SemaphoreType.DMA,
        ],
    )
    def k(feat_hbm, idx_hbm, out_p_hbm, out_n_hbm,
          idx_v, rows_v, acc_v, sem_g, sem_o):
        wid = lax.axis_index("s") * nc + lax.axis_index("c")
        pltpu.sync_copy(idx_hbm.at[wid], idx_v)

        nsteps = 2 * S_STEPS

        def gather_args(t):
            s, pn, slot = t // 2, t % 2, t % 2
            return [(feat_hbm.at[idx_v.at[s, pn, h]],
                     rows_v.at[slot, pl.ds(h * RPS, RPS)], sem_g)
                    for h in range(NSTREAM)]

        def issue(t):
            for a in gather_args(t):
                pltpu.async_copy(*a)

        def drain(t):
            for a in gather_args(t):
                pltpu.make_async_copy(*a).wait()

        def out_args(p):
            par = p % 2
            base = wid * (S_STEPS * GN) + p * FN
            return [(acc_v.at[par, 0], out_p_hbm.at[pl.ds(base, FN)], sem_o),
                    (acc_v.at[par, 1], out_n_hbm.at[pl.ds(base, FN)], sem_o)]

        issue(0)

        def body(t, _):
            s, pn, slot = t // 2, t % 2, t % 2
            p = s // 2
            par = p % 2
            off = (s % 2) * GN

            @pl.when(t + 1 < nsteps)
            def _():
                issue(t + 1)

            @pl.when((pn == 0) & (s % 2 == 0) & (p >= 2))
            def _():
                for a in out_args(p - 2):
                    pltpu.make_async_copy(*a).wait()

            drain(t)

            def grp(g, _):
                for c in range(D // 16):
                    sl = pl.ds(c * 16, 16)
                    a = rows_v[slot, g * K + 0, sl]
                    for j in range(1, K):
                        a = a + rows_v[slot, g * K + j, sl]
                    acc_v[par, pn, off + g, sl] = a
                return 0

            lax.fori_loop(0, GN, grp, 0, unroll=False)

            @pl.when((pn == 1) & (s % 2 == 1))
            def _():
                for a in out_args(p):
                    pltpu.async_copy(*a)

            return 0

        lax.fori_loop(0, nsteps, body, 0, unroll=False)

        nflush = S_STEPS // 2
        for p in (nflush - 2, nflush - 1):
            for a in out_args(p):
                pltpu.make_async_copy(*a).wait()

    return k(features, idx)


def _tc_project(s_pos, s_neg, w_bal, w_unbal):
    blk = 512
    grid = (pl.cdiv(B, blk),)
    dn = (((1,), (1,)), ((), ()))

    def body(sp_ref, sn_ref, wb_ref, wu_ref, ob_ref, ou_ref):
        scale = jnp.float32(1.0 / K)
        ob_ref[...] = jnp.tanh(scale * lax.dot_general(
            wb_ref[...], sp_ref[...], dn, preferred_element_type=jnp.float32))
        ou_ref[...] = jnp.tanh(scale * lax.dot_general(
            wu_ref[...], sn_ref[...], dn, preferred_element_type=jnp.float32))

    return pl.pallas_call(
        body,
        grid=grid,
        in_specs=[
            pl.BlockSpec((blk, D), lambda i: (i, 0)),
            pl.BlockSpec((blk, D), lambda i: (i, 0)),
            pl.BlockSpec((E, D), lambda i: (0, 0)),
            pl.BlockSpec((E, D), lambda i: (0, 0)),
        ],
        out_specs=[
            pl.BlockSpec((E, blk), lambda i: (0, i)),
            pl.BlockSpec((E, blk), lambda i: (0, i)),
        ],
        out_shape=[
            jax.ShapeDtypeStruct((E, B), jnp.float32),
            jax.ShapeDtypeStruct((E, B), jnp.float32),
        ],
    )(s_pos, s_neg, w_bal, w_unbal)


def kernel(nodes, pos_neigh, neg_neigh, features, W_bal, W_unbal):
    idx = _prep_idx(pos_neigh, neg_neigh)
    s_pos, s_neg = _sc_gather_sum(features, idx)
    mapped_bal, mapped_unbal = _tc_project(s_pos, s_neg, W_bal, W_unbal)
    return (mapped_bal, mapped_unbal)


# speedup vs baseline: 1.7407x; 1.7407x over previous
import functools

import numpy as np

import jax
import jax.numpy as jnp
from jax import lax
from jax.experimental import pallas as pl
from jax.experimental.pallas import tpu as pltpu
from jax.experimental.pallas import tpu_sc as plsc

N_NODES = 50000
B = 50000
K = 10
D = 128
E = 128
NW = 32
GN = 16
NSTREAM = 2
S_STEPS = 98
B_PAD = NW * S_STEPS * GN

RPS = GN * K // NSTREAM

_SIDX = np.repeat(np.arange(GN, dtype=np.int32), K).reshape(NSTREAM, RPS)


def _prep_idx(pos_neigh, neg_neigh):
    def one(neigh):
        flat = neigh.astype(jnp.int32).reshape(-1)
        flat = jnp.pad(flat, (0, B_PAD * K - B * K))
        return flat.reshape(NW, S_STEPS, 1, NSTREAM, RPS)
    return jnp.concatenate([one(pos_neigh), one(neg_neigh)], axis=2)


def _sc_gather_sum(features, idx, sidx):
    info = plsc.get_sparse_core_info()
    nc = info.num_cores

    mesh = plsc.VectorSubcoreMesh(core_axis_name="c", subcore_axis_name="s")

    @functools.partial(
        pl.kernel,
        out_type=(jax.ShapeDtypeStruct((B_PAD, D), jnp.float32),
                  jax.ShapeDtypeStruct((B_PAD, D), jnp.float32)),
        mesh=mesh,
        scratch_types=[
            pltpu.VMEM((S_STEPS, 2, NSTREAM, RPS), jnp.int32),
            pltpu.VMEM((NSTREAM, RPS), jnp.int32),
            pltpu.VMEM((2, GN * K, D), jnp.float32),
            pltpu.VMEM_SHARED((16, 2, 2, GN, D), jnp.float32),
            pltpu.VMEM((GN, D), jnp.float32),
            pltpu.SemaphoreType.DMA,
            pltpu.SemaphoreType.DMA,
            pltpu.SemaphoreType.DMA,
            pltpu.SemaphoreType.DMA,
        ],
    )
    def k(feat_hbm, idx_hbm, sidx_hbm, out_p_hbm, out_n_hbm,
          idx_v, sidx_v, rows_v, acc_sh, zeros_v, sem_g, sem_a, sem_o, sem_z):
        sid = lax.axis_index("s")
        wid = sid * nc + lax.axis_index("c")
        pltpu.sync_copy(idx_hbm.at[wid], idx_v)
        pltpu.sync_copy(sidx_hbm, sidx_v)

        def zv(g, _):
            for c in range(D // 16):
                zeros_v[g, pl.ds(c * 16, 16)] = jnp.zeros((16,), jnp.float32)
            return 0

        lax.fori_loop(0, GN, zv, 0, unroll=False)

        nsteps = 2 * S_STEPS

        def gather_args(t):
            s, pn, slot = t // 2, t % 2, t % 2
            return [(feat_hbm.at[idx_v.at[s, pn, h]],
                     rows_v.at[slot, pl.ds(h * RPS, RPS)], sem_g)
                    for h in range(NSTREAM)]

        def issue(t):
            for a in gather_args(t):
                pltpu.async_copy(*a)

        def drain(t):
            for a in gather_args(t):
                pltpu.make_async_copy(*a).wait()

        def out_args(s):
            par = s % 2
            base = wid * (S_STEPS * GN) + s * GN
            return [(acc_sh.at[sid, par, 0], out_p_hbm.at[pl.ds(base, GN)], sem_o),
                    (acc_sh.at[sid, par, 1], out_n_hbm.at[pl.ds(base, GN)], sem_o)]

        issue(0)

        def body(t, _):
            s, pn, slot = t // 2, t % 2, t % 2
            par = s % 2

            @pl.when(t + 1 < nsteps)
            def _():
                issue(t + 1)

            @pl.when((pn == 0) & (s >= 2))
            def _():
                for a in out_args(s - 2):
                    pltpu.make_async_copy(*a).wait()

            pltpu.async_copy(zeros_v, acc_sh.at[sid, par, pn], sem_z).wait()

            drain(t)

            for h in range(NSTREAM):
                pltpu.async_copy(
                    rows_v.at[slot, pl.ds(h * RPS, RPS)],
                    acc_sh.at[sid, par, pn].at[sidx_v.at[h]],
                    sem_a, add=True).wait()

            @pl.when(pn == 1)
            def _():
                for a in out_args(s):
                    pltpu.async_copy(*a)

            return 0

        lax.fori_loop(0, nsteps, body, 0, unroll=False)

        for s in (S_STEPS - 2, S_STEPS - 1):
            for a in out_args(s):
                pltpu.make_async_copy(*a).wait()

    return k(features, idx, sidx)


def _tc_project(s_pos, s_neg, w_bal, w_unbal):
    blk = 512
    grid = (pl.cdiv(B, blk),)
    dn = (((1,), (1,)), ((), ()))

    def body(sp_ref, sn_ref, wb_ref, wu_ref, ob_ref, ou_ref):
        scale = jnp.float32(1.0 / K)
        ob_ref[...] = jnp.tanh(scale * lax.dot_general(
            wb_ref[...], sp_ref[...], dn, preferred_element_type=jnp.float32))
        ou_ref[...] = jnp.tanh(scale * lax.dot_general(
            wu_ref[...], sn_ref[...], dn, preferred_element_type=jnp.float32))

    return pl.pallas_call(
        body,
        grid=grid,
        in_specs=[
            pl.BlockSpec((blk, D), lambda i: (i, 0)),
            pl.BlockSpec((blk, D), lambda i: (i, 0)),
            pl.BlockSpec((E, D), lambda i: (0, 0)),
            pl.BlockSpec((E, D), lambda i: (0, 0)),
        ],
        out_specs=[
            pl.BlockSpec((E, blk), lambda i: (0, i)),
            pl.BlockSpec((E, blk), lambda i: (0, i)),
        ],
        out_shape=[
            jax.ShapeDtypeStruct((E, B), jnp.float32),
            jax.ShapeDtypeStruct((E, B), jnp.float32),
        ],
    )(s_pos, s_neg, w_bal, w_unbal)


def kernel(nodes, pos_neigh, neg_neigh, features, W_bal, W_unbal):
    idx = _prep_idx(pos_neigh, neg_neigh)
    s_pos, s_neg = _sc_gather_sum(features, idx, jnp.asarray(_SIDX))
    mapped_bal, mapped_unbal = _tc_project(s_pos, s_neg, W_bal, W_unbal)
    return (mapped_bal, mapped_unbal)
